# Initial kernel scaffold; baseline (speedup 1.0000x reference)
import jax
import jax.numpy as jnp
from jax.experimental import pallas as pl


def kernel(x, gate_w, w1, wg, w2):
    raise NotImplementedError("write your pallas kernel here")


# speedup vs baseline: 1.2483x; 1.2483x over previous
import functools

import jax
import jax.numpy as jnp
from jax import lax
from jax.experimental import pallas as pl
from jax.experimental.pallas import tpu as pltpu
from jax.experimental.pallas import tpu_sc as plsc

B, S, D = 2, 2048, 2048
E, TK, H = 8, 2, 1024
T = B * S
NPAIR = T * TK
BLK = 256
P = NPAIR + E * BLK
NB = P // BLK
NBPAD = 48
HB = 256
NH = H // HB

NC, NS = 2, 16
NW = NC * NS
PW = NPAIR // NW
TW = T // NW

_MESH = plsc.VectorSubcoreMesh(
    core_axis_name="c", subcore_axis_name="s", num_cores=NC, num_subcores=NS)


def _wid():
    return lax.axis_index("s") * NC + lax.axis_index("c")


def _router_body(x_ref, g_ref, topi_ref, topv_ref):
    xb = x_ref[...]
    gw = g_ref[...]
    logits = lax.dot_general(xb, gw, (((1,), (1,)), ((), ())),
                             preferred_element_type=jnp.float32)
    m = jnp.max(logits, axis=-1, keepdims=True)
    p = jnp.exp(logits - m)
    p = p / jnp.sum(p, axis=-1, keepdims=True)
    i8 = lax.broadcasted_iota(jnp.int32, p.shape, 1)
    m1 = jnp.max(p, axis=-1, keepdims=True)
    i1 = jnp.min(jnp.where(p == m1, i8, E + 1), axis=-1, keepdims=True)
    p2 = jnp.where(i8 == i1, -1.0, p)
    m2 = jnp.max(p2, axis=-1, keepdims=True)
    i2 = jnp.min(jnp.where(p2 == m2, i8, E + 1), axis=-1, keepdims=True)
    s = m1 + m2
    topi_ref[...] = jnp.concatenate([i1, i2], axis=1)
    topv_ref[...] = jnp.concatenate([m1 / s, m2 / s], axis=1)


def _router(x2, gate_w):
    rb = 256
    return pl.pallas_call(
        _router_body,
        grid=(T // rb,),
        in_specs=[
            pl.BlockSpec((rb, D), lambda i: (i, 0)),
            pl.BlockSpec((E, D), lambda i: (0, 0)),
        ],
        out_specs=[
            pl.BlockSpec((rb, TK), lambda i: (i, 0)),
            pl.BlockSpec((rb, TK), lambda i: (i, 0)),
        ],
        out_shape=[
            jax.ShapeDtypeStruct((T, TK), jnp.int32),
            jax.ShapeDtypeStruct((T, TK), jnp.float32),
        ],
    )(x2, gate_w)


def _hist_body(topi_hbm, hist_hbm, buf, hist_v):
    w = _wid()
    pltpu.sync_copy(topi_hbm.at[pl.ds(w * PW, PW)], buf)
    hist_v[...] = jnp.zeros((16,), jnp.int32)
    ones = jnp.ones((16,), jnp.int32)
    for i in range(PW // 16):
        v = buf[pl.ds(i * 16, 16)]
        plsc.addupdate_scatter(hist_v, [v], ones)
    pltpu.sync_copy(hist_v, hist_hbm.at[w])


def _hist(topi_flat):
    return pl.kernel(
        _hist_body,
        out_type=jax.ShapeDtypeStruct((NW, 16), jnp.int32),
        mesh=_MESH,
        compiler_params=pltpu.CompilerParams(needs_layout_passes=False),
        scratch_types=[
            pltpu.VMEM((PW,), jnp.int32),
            pltpu.VMEM((16,), jnp.int32),
        ],
    )(topi_flat)


def _dispatch_body(topi_hbm, hist_hbm, pos_hbm, blk_hbm,
                   buf, hist_v, base_v, cur_v, pos_v, offblk_v, blk_v):
    w = _wid()
    pltpu.sync_copy(topi_hbm.at[pl.ds(w * PW, PW)], buf)
    pltpu.sync_copy(hist_hbm, hist_v)
    zeros = jnp.zeros((16,), jnp.int32)
    ones = jnp.ones((16,), jnp.int32)
    tot = zeros
    prior = zeros
    for i in range(NW):
        row = hist_v[i, :]
        tot = tot + row
        mask = jnp.full((16,), i, jnp.int32) < w
        prior = prior + jnp.where(mask, row, 0)
    pad = (tot + (BLK - 1)) & ~(BLK - 1)
    offpad = plsc.cumsum(pad) - pad
    base_v[...] = offpad + prior
    cur_v[...] = zeros
    for i in range(PW // 16):
        v = buf[pl.ds(i * 16, 16)]
        gb = plsc.load_gather(base_v, [v])
        gc = plsc.load_gather(cur_v, [v])
        plsc.addupdate_scatter(cur_v, [v], ones)
        rank = zeros
        for e in range(E):
            m = v == e
            r = plsc.cumsum(jnp.where(m, 1, 0))
            rank = jnp.where(m, r - 1, rank)
        pos_v[pl.ds(i * 16, 16)] = gb + gc + rank
    pltpu.sync_copy(pos_v, pos_hbm.at[pl.ds(w * PW, PW)])

    @pl.when(w == 0)
    def _():
        offblk_v[...] = offpad // BLK
        for c in range(NBPAD // 16):
            bvec = lax.iota(jnp.int32, 16) + c * 16
            acc = jnp.zeros((16,), jnp.int32)
            for e in range(1, E):
                s_e = plsc.load_gather(offblk_v, [jnp.full((16,), e, jnp.int32)])
                acc = acc + jnp.where(bvec >= s_e, 1, 0)
            blk_v[pl.ds(c * 16, 16)] = acc
        pltpu.sync_copy(blk_v, blk_hbm)


def _dispatch(topi_flat, hist):
    return pl.kernel(
        _dispatch_body,
        out_type=[
            jax.ShapeDtypeStruct((NPAIR,), jnp.int32),
            jax.ShapeDtypeStruct((NBPAD,), jnp.int32),
        ],
        mesh=_MESH,
        compiler_params=pltpu.CompilerParams(needs_layout_passes=False),
        scratch_types=[
            pltpu.VMEM((PW,), jnp.int32),
            pltpu.VMEM((NW, 16), jnp.int32),
            pltpu.VMEM((16,), jnp.int32),
            pltpu.VMEM((16,), jnp.int32),
            pltpu.VMEM((PW,), jnp.int32),
            pltpu.VMEM((16,), jnp.int32),
            pltpu.VMEM((NBPAD,), jnp.int32),
        ],
    )(topi_flat, hist)


_XCH = 16

def _xscat_body(x_hbm, pos_hbm, xs_hbm, pos_v, xrows, idx_e, idx_o, sem):
    w = _wid()
    pltpu.sync_copy(pos_hbm.at[pl.ds(w * PW, PW)], pos_v)
    ii = lax.iota(jnp.int32, 16)
    for c in range(TW // _XCH):
        tok0 = w * TW + c * _XCH
        pltpu.sync_copy(x_hbm.at[pl.ds(tok0, _XCH)], xrows)
        idx_e[...] = plsc.load_gather(pos_v, [c * 2 * _XCH + 2 * ii])
        idx_o[...] = plsc.load_gather(pos_v, [c * 2 * _XCH + 2 * ii + 1])
        cp1 = pltpu.async_copy(xrows, xs_hbm.at[idx_e], sem)
        cp2 = pltpu.async_copy(xrows, xs_hbm.at[idx_o], sem)
        cp1.wait()
        cp2.wait()


def _xscatter(x2, pos):
    return pl.kernel(
        _xscat_body,
        out_type=jax.ShapeDtypeStruct((P, D), jnp.float32),
        mesh=_MESH,
        compiler_params=pltpu.CompilerParams(needs_layout_passes=False),
        scratch_types=[
            pltpu.VMEM((PW,), jnp.int32),
            pltpu.VMEM((_XCH, D), jnp.float32),
            pltpu.VMEM((16,), jnp.int32),
            pltpu.VMEM((16,), jnp.int32),
            pltpu.SemaphoreType.DMA,
        ],
    )(x2, pos)


def _ffn_body(be_ref, xs_ref, wg_ref, w1_ref, w2_ref, eo_ref):
    k = pl.program_id(1)
    xb = xs_ref[...]
    g = lax.dot_general(xb, wg_ref[0], (((1,), (1,)), ((), ())),
                        preferred_element_type=jnp.float32)
    a = lax.dot_general(xb, w1_ref[0], (((1,), (1,)), ((), ())),
                        preferred_element_type=jnp.float32)
    h = a * (g * jax.nn.sigmoid(g))
    contrib = lax.dot_general(h, w2_ref[0], (((1,), (1,)), ((), ())),
                              preferred_element_type=jnp.float32)

    @pl.when(k == 0)
    def _():
        eo_ref[...] = contrib

    @pl.when(k > 0)
    def _():
        eo_ref[...] = eo_ref[...] + contrib


def _ffn(blkexp, xs, wg, w1, w2):
    grid_spec = pltpu.PrefetchScalarGridSpec(
        num_scalar_prefetch=1,
        grid=(NB, NH),
        in_specs=[
            pl.BlockSpec((BLK, D), lambda b, k, be: (b, 0)),
            pl.BlockSpec((1, HB, D), lambda b, k, be: (be[b], k, 0)),
            pl.BlockSpec((1, HB, D), lambda b, k, be: (be[b], k, 0)),
            pl.BlockSpec((1, D, HB), lambda b, k, be: (be[b], 0, k)),
        ],
        out_specs=pl.BlockSpec((BLK, D), lambda b, k, be: (b, 0)),
    )
    return pl.pallas_call(
        _ffn_body,
        grid_spec=grid_spec,
        out_shape=jax.ShapeDtypeStruct((P, D), jnp.float32),
        compiler_params=pltpu.CompilerParams(
            dimension_semantics=("arbitrary", "arbitrary")),
    )(blkexp, xs, wg, w1, w2)


_CCH = 8

def _comb_body(eo_hbm, pos_hbm, tv_hbm, out_hbm, pos_v, tv_v, idx, rows, orow,
               sem):
    w = _wid()
    pltpu.sync_copy(pos_hbm.at[pl.ds(w * PW, PW)], pos_v)
    pltpu.sync_copy(tv_hbm.at[pl.ds(w * PW, PW)], tv_v)
    for c in range(TW // _CCH):
        idx[...] = pos_v[pl.ds(c * 2 * _CCH, 2 * _CCH)]
        pltpu.async_copy(eo_hbm.at[idx], rows, sem).wait()
        wv = tv_v[pl.ds(c * 2 * _CCH, 2 * _CCH)]

        def body_r(r, _):
            off = r * 16
            for i in range(_CCH):
                a = rows[2 * i, pl.ds(off, 16)]
                b = rows[2 * i + 1, pl.ds(off, 16)]
                orow[i, pl.ds(off, 16)] = a * wv[2 * i] + b * wv[2 * i + 1]
            return 0

        lax.fori_loop(0, D // 16, body_r, 0)
        pltpu.sync_copy(orow, out_hbm.at[pl.ds(w * TW + c * _CCH, _CCH)])


def _combine(eo, pos, tv_flat):
    return pl.kernel(
        _comb_body,
        out_type=jax.ShapeDtypeStruct((T, D), jnp.float32),
        mesh=_MESH,
        compiler_params=pltpu.CompilerParams(needs_layout_passes=False),
        scratch_types=[
            pltpu.VMEM((PW,), jnp.int32),
            pltpu.VMEM((PW,), jnp.float32),
            pltpu.VMEM((2 * _CCH,), jnp.int32),
            pltpu.VMEM((2 * _CCH, D), jnp.float32),
            pltpu.VMEM((_CCH, D), jnp.float32),
            pltpu.SemaphoreType.DMA,
        ],
    )(eo, pos, tv_flat)


def kernel(x, gate_w, w1, wg, w2):
    x2 = x.reshape(T, D)
    topi, topv = _router(x2, gate_w)
    topi_flat = topi.reshape(NPAIR)
    tv_flat = topv.reshape(NPAIR)
    hist = _hist(topi_flat)
    pos, blkexp = _dispatch(topi_flat, hist)
    xs = _xscatter(x2, pos)
    eo = _ffn(blkexp, xs, wg, w1, w2)
    out2 = _combine(eo, pos, tv_flat)
    return out2.reshape(B, S, D)


# speedup vs baseline: 1.8841x; 1.5094x over previous
import functools

import jax
import jax.numpy as jnp
from jax import lax
from jax.experimental import pallas as pl
from jax.experimental.pallas import tpu as pltpu
from jax.experimental.pallas import tpu_sc as plsc

B, S, D = 2, 2048, 2048
E, TK, H = 8, 2, 1024
T = B * S
NPAIR = T * TK
BLK = 256
P = NPAIR + E * BLK
NB = P // BLK
NBPAD = 48
HB = 256
NH = H // HB

NC, NS = 2, 16
NW = NC * NS
PW = NPAIR // NW
TW = T // NW

_MESH = plsc.VectorSubcoreMesh(
    core_axis_name="c", subcore_axis_name="s", num_cores=NC, num_subcores=NS)


def _wid():
    return lax.axis_index("s") * NC + lax.axis_index("c")


def _router_body(x_ref, g_ref, topi_ref, topv_ref):
    xb = x_ref[...]
    gw = g_ref[...]
    logits = lax.dot_general(xb, gw, (((1,), (1,)), ((), ())),
                             preferred_element_type=jnp.float32)
    m = jnp.max(logits, axis=-1, keepdims=True)
    p = jnp.exp(logits - m)
    p = p / jnp.sum(p, axis=-1, keepdims=True)
    i8 = lax.broadcasted_iota(jnp.int32, p.shape, 1)
    m1 = jnp.max(p, axis=-1, keepdims=True)
    i1 = jnp.min(jnp.where(p == m1, i8, E + 1), axis=-1, keepdims=True)
    p2 = jnp.where(i8 == i1, -1.0, p)
    m2 = jnp.max(p2, axis=-1, keepdims=True)
    i2 = jnp.min(jnp.where(p2 == m2, i8, E + 1), axis=-1, keepdims=True)
    s = m1 + m2
    topi_ref[...] = jnp.concatenate([i1, i2], axis=1)
    topv_ref[...] = jnp.concatenate([m1 / s, m2 / s], axis=1)


def _router(x2, gate_w):
    rb = 256
    return pl.pallas_call(
        _router_body,
        grid=(T // rb,),
        in_specs=[
            pl.BlockSpec((rb, D), lambda i: (i, 0)),
            pl.BlockSpec((E, D), lambda i: (0, 0)),
        ],
        out_specs=[
            pl.BlockSpec((rb, TK), lambda i: (i, 0)),
            pl.BlockSpec((rb, TK), lambda i: (i, 0)),
        ],
        out_shape=[
            jax.ShapeDtypeStruct((T, TK), jnp.int32),
            jax.ShapeDtypeStruct((T, TK), jnp.float32),
        ],
    )(x2, gate_w)


def _hist_body(topi_hbm, hist_hbm, buf, hist_v):
    w = _wid()
    pltpu.sync_copy(topi_hbm.at[pl.ds(w * PW, PW)], buf)
    hist_v[...] = jnp.zeros((16,), jnp.int32)
    ones = jnp.ones((16,), jnp.int32)
    for i in range(PW // 16):
        v = buf[pl.ds(i * 16, 16)]
        plsc.addupdate_scatter(hist_v, [v], ones)
    pltpu.sync_copy(hist_v, hist_hbm.at[w])


def _hist(topi_flat):
    return pl.kernel(
        _hist_body,
        out_type=jax.ShapeDtypeStruct((NW, 16), jnp.int32),
        mesh=_MESH,
        compiler_params=pltpu.CompilerParams(needs_layout_passes=False),
        scratch_types=[
            pltpu.VMEM((PW,), jnp.int32),
            pltpu.VMEM((16,), jnp.int32),
        ],
    )(topi_flat)


def _dispatch_body(topi_hbm, hist_hbm, pos_hbm, blk_hbm,
                   buf, hist_v, base_v, cur_v, pos_v, offblk_v, blk_v):
    w = _wid()
    pltpu.sync_copy(topi_hbm.at[pl.ds(w * PW, PW)], buf)
    pltpu.sync_copy(hist_hbm, hist_v)
    zeros = jnp.zeros((16,), jnp.int32)
    ones = jnp.ones((16,), jnp.int32)
    tot = zeros
    prior = zeros
    for i in range(NW):
        row = hist_v[i, :]
        tot = tot + row
        mask = jnp.full((16,), i, jnp.int32) < w
        prior = prior + jnp.where(mask, row, 0)
    pad = (tot + (BLK - 1)) & ~(BLK - 1)
    offpad = plsc.cumsum(pad) - pad
    base_v[...] = offpad + prior
    cur_v[...] = zeros
    for i in range(PW // 16):
        v = buf[pl.ds(i * 16, 16)]
        gb = plsc.load_gather(base_v, [v])
        gc = plsc.load_gather(cur_v, [v])
        plsc.addupdate_scatter(cur_v, [v], ones)
        rank = zeros
        for e in range(E):
            m = v == e
            r = plsc.cumsum(jnp.where(m, 1, 0))
            rank = jnp.where(m, r - 1, rank)
        pos_v[pl.ds(i * 16, 16)] = gb + gc + rank
    pltpu.sync_copy(pos_v, pos_hbm.at[pl.ds(w * PW, PW)])

    @pl.when(w == 0)
    def _():
        offblk_v[...] = offpad // BLK
        for c in range(NBPAD // 16):
            bvec = lax.iota(jnp.int32, 16) + c * 16
            acc = jnp.zeros((16,), jnp.int32)
            for e in range(1, E):
                s_e = plsc.load_gather(offblk_v, [jnp.full((16,), e, jnp.int32)])
                acc = acc + jnp.where(bvec >= s_e, 1, 0)
            blk_v[pl.ds(c * 16, 16)] = acc
        pltpu.sync_copy(blk_v, blk_hbm)


def _dispatch(topi_flat, hist):
    return pl.kernel(
        _dispatch_body,
        out_type=[
            jax.ShapeDtypeStruct((NPAIR,), jnp.int32),
            jax.ShapeDtypeStruct((NBPAD,), jnp.int32),
        ],
        mesh=_MESH,
        compiler_params=pltpu.CompilerParams(needs_layout_passes=False),
        scratch_types=[
            pltpu.VMEM((PW,), jnp.int32),
            pltpu.VMEM((NW, 16), jnp.int32),
            pltpu.VMEM((16,), jnp.int32),
            pltpu.VMEM((16,), jnp.int32),
            pltpu.VMEM((PW,), jnp.int32),
            pltpu.VMEM((16,), jnp.int32),
            pltpu.VMEM((NBPAD,), jnp.int32),
        ],
    )(topi_flat, hist)


_XCH = 16

def _xscat_body(x_hbm, pos_hbm, xs_hbm, pos_v, xrows, idx_e, idx_o, sem):
    w = _wid()
    pltpu.sync_copy(pos_hbm.at[pl.ds(w * PW, PW)], pos_v)
    ii = lax.iota(jnp.int32, 16)
    for c in range(TW // _XCH):
        tok0 = w * TW + c * _XCH
        pltpu.sync_copy(x_hbm.at[pl.ds(tok0, _XCH)], xrows)
        idx_e[...] = plsc.load_gather(pos_v, [c * 2 * _XCH + 2 * ii])
        idx_o[...] = plsc.load_gather(pos_v, [c * 2 * _XCH + 2 * ii + 1])
        cp1 = pltpu.async_copy(xrows, xs_hbm.at[idx_e], sem)
        cp2 = pltpu.async_copy(xrows, xs_hbm.at[idx_o], sem)
        cp1.wait()
        cp2.wait()


def _xscatter(x2, pos):
    return pl.kernel(
        _xscat_body,
        out_type=jax.ShapeDtypeStruct((P, D), jnp.float32),
        mesh=_MESH,
        compiler_params=pltpu.CompilerParams(needs_layout_passes=False),
        scratch_types=[
            pltpu.VMEM((PW,), jnp.int32),
            pltpu.VMEM((_XCH, D), jnp.float32),
            pltpu.VMEM((16,), jnp.int32),
            pltpu.VMEM((16,), jnp.int32),
            pltpu.SemaphoreType.DMA,
        ],
    )(x2, pos)


def _ffn_body(be_ref, xs_ref, wg_ref, w1_ref, w2_ref, eo_ref):
    xb = xs_ref[...]
    g = lax.dot_general(xb, wg_ref[0], (((1,), (1,)), ((), ())),
                        preferred_element_type=jnp.float32)
    a = lax.dot_general(xb, w1_ref[0], (((1,), (1,)), ((), ())),
                        preferred_element_type=jnp.float32)
    h = a * (g * jax.nn.sigmoid(g))
    eo_ref[...] = lax.dot_general(h, w2_ref[0], (((1,), (1,)), ((), ())),
                                  preferred_element_type=jnp.float32)


def _ffn(blkexp, xs, wg, w1, w2):
    grid_spec = pltpu.PrefetchScalarGridSpec(
        num_scalar_prefetch=1,
        grid=(NB,),
        in_specs=[
            pl.BlockSpec((BLK, D), lambda b, be: (b, 0)),
            pl.BlockSpec((1, H, D), lambda b, be: (be[b], 0, 0)),
            pl.BlockSpec((1, H, D), lambda b, be: (be[b], 0, 0)),
            pl.BlockSpec((1, D, H), lambda b, be: (be[b], 0, 0)),
        ],
        out_specs=pl.BlockSpec((BLK, D), lambda b, be: (b, 0)),
    )
    return pl.pallas_call(
        _ffn_body,
        grid_spec=grid_spec,
        out_shape=jax.ShapeDtypeStruct((P, D), jnp.float32),
        compiler_params=pltpu.CompilerParams(
            dimension_semantics=("arbitrary",),
            vmem_limit_bytes=100 * 1024 * 1024),
    )(blkexp, xs, wg, w1, w2)


_CCH = 8

def _comb_body(eo_hbm, pos_hbm, tv_hbm, out_hbm, pos_v, tv_v, idx, rows, orow,
               sem):
    w = _wid()
    pltpu.sync_copy(pos_hbm.at[pl.ds(w * PW, PW)], pos_v)
    pltpu.sync_copy(tv_hbm.at[pl.ds(w * PW, PW)], tv_v)
    for c in range(TW // _CCH):
        idx[...] = pos_v[pl.ds(c * 2 * _CCH, 2 * _CCH)]
        pltpu.async_copy(eo_hbm.at[idx], rows, sem).wait()
        wv = tv_v[pl.ds(c * 2 * _CCH, 2 * _CCH)]

        def body_r(r, _):
            off = r * 16
            for i in range(_CCH):
                a = rows[2 * i, pl.ds(off, 16)]
                b = rows[2 * i + 1, pl.ds(off, 16)]
                orow[i, pl.ds(off, 16)] = a * wv[2 * i] + b * wv[2 * i + 1]
            return 0

        lax.fori_loop(0, D // 16, body_r, 0)
        pltpu.sync_copy(orow, out_hbm.at[pl.ds(w * TW + c * _CCH, _CCH)])


def _combine(eo, pos, tv_flat):
    return pl.kernel(
        _comb_body,
        out_type=jax.ShapeDtypeStruct((T, D), jnp.float32),
        mesh=_MESH,
        compiler_params=pltpu.CompilerParams(needs_layout_passes=False),
        scratch_types=[
            pltpu.VMEM((PW,), jnp.int32),
            pltpu.VMEM((PW,), jnp.float32),
            pltpu.VMEM((2 * _CCH,), jnp.int32),
            pltpu.VMEM((2 * _CCH, D), jnp.float32),
            pltpu.VMEM((_CCH, D), jnp.float32),
            pltpu.SemaphoreType.DMA,
        ],
    )(eo, pos, tv_flat)


def kernel(x, gate_w, w1, wg, w2):
    x2 = x.reshape(T, D)
    topi, topv = _router(x2, gate_w)
    topi_flat = topi.reshape(NPAIR)
    tv_flat = topv.reshape(NPAIR)
    hist = _hist(topi_flat)
    pos, blkexp = _dispatch(topi_flat, hist)
    xs = _xscatter(x2, pos)
    eo = _ffn(blkexp, xs, wg, w1, w2)
    out2 = _combine(eo, pos, tv_flat)
    return out2.reshape(B, S, D)
